# baseline (speedup 1.0000x reference)
import functools

import jax
import jax.numpy as jnp
from jax.experimental import pallas as pl
from jax.experimental.pallas import tpu as pltpu


def _fold(logits, tile_idx, vt, v, max_sc, idx_sc):
    col = tile_idx * vt + jax.lax.broadcasted_iota(jnp.int32, logits.shape, 1)
    logits = jnp.where(col < v, logits, -jnp.inf)

    local_max = jnp.max(logits, axis=1, keepdims=True)
    local_idx = (jnp.argmax(logits, axis=1).astype(jnp.int32)[:, None]
                 + tile_idx * vt)

    better = local_max > max_sc[...]
    idx_sc[...] = jnp.where(better, local_idx, idx_sc[...])
    max_sc[...] = jnp.where(better, local_max, max_sc[...])


def _argmax_matmul_kernel(pos_ref, x_ref, emb_a_ref, emb_b_ref, out_ref,
                          max_sc, idx_sc, *, vt: int, ng: int, v: int):
    i = pl.program_id(0)

    @pl.when(i == 0)
    def _init():
        max_sc[...] = jnp.full_like(max_sc[...], -jnp.inf)
        idx_sc[...] = jnp.zeros_like(idx_sc[...])

    xs = x_ref[...]
    dims = (((1,), (1,)), ((), ()))
    logits_a = jax.lax.dot_general(xs, emb_a_ref[...], dims,
                                   preferred_element_type=jnp.float32)
    _fold(logits_a, 2 * i, vt, v, max_sc, idx_sc)
    logits_b = jax.lax.dot_general(xs, emb_b_ref[...], dims,
                                   preferred_element_type=jnp.float32)
    _fold(logits_b, 2 * i + 1, vt, v, max_sc, idx_sc)

    @pl.when(i == ng - 1)
    def _done():
        out_ref[...] = idx_sc[...]


def kernel(embedding, x, output_pos, temperature, topp, topk, embedding_bias=None):
    v, d = embedding.shape
    b, s, _ = x.shape
    vt = 2048
    ng = pl.cdiv(v, 2 * vt)
    last_tile = pl.cdiv(v, vt) - 1

    xt = x.reshape(b, s * d)
    pos = output_pos.astype(jnp.int32)

    grid_spec = pltpu.PrefetchScalarGridSpec(
        num_scalar_prefetch=1,
        grid=(ng,),
        in_specs=[
            pl.BlockSpec((b, d), lambda i, pos_ref: (0, pos_ref[0])),
            pl.BlockSpec((vt, d),
                         lambda i, pos_ref: (jnp.minimum(2 * i, last_tile), 0)),
            pl.BlockSpec((vt, d),
                         lambda i, pos_ref: (jnp.minimum(2 * i + 1, last_tile), 0)),
        ],
        out_specs=pl.BlockSpec((b, 1), lambda i, pos_ref: (0, 0)),
        scratch_shapes=[
            pltpu.VMEM((b, 1), jnp.float32),
            pltpu.VMEM((b, 1), jnp.int32),
        ],
    )
    out = pl.pallas_call(
        functools.partial(_argmax_matmul_kernel, vt=vt, ng=ng, v=v),
        grid_spec=grid_spec,
        out_shape=jax.ShapeDtypeStruct((b, 1), jnp.int32),
        compiler_params=pltpu.CompilerParams(
            vmem_limit_bytes=100 * 1024 * 1024),
    )(pos, xt, embedding, embedding)
    return out[:, 0]


# speedup vs baseline: 1.0126x; 1.0126x over previous
import functools

import jax
import jax.numpy as jnp
from jax.experimental import pallas as pl
from jax.experimental.pallas import tpu as pltpu


def _fold(logits, tile_idx, vt, max_sc, idx_sc):
    local_max = jnp.max(logits, axis=1, keepdims=True)
    local_idx = (jnp.argmax(logits, axis=1).astype(jnp.int32)[:, None]
                 + tile_idx * vt)
    better = local_max > max_sc[...]
    idx_sc[...] = jnp.where(better, local_idx, idx_sc[...])
    max_sc[...] = jnp.where(better, local_max, max_sc[...])


def _argmax_matmul_kernel(pos_ref, x_ref, emb_ref, out_ref,
                          logits_sc, max_sc, idx_sc, *, vt: int, ng: int):
    i = pl.program_id(0)
    p = jax.lax.rem(i, 2)

    @pl.when(i == 0)
    def _init():
        max_sc[...] = jnp.full_like(max_sc[...], -jnp.inf)
        idx_sc[...] = jnp.zeros_like(idx_sc[...])

    @pl.when(i > 0)
    def _fold_prev():
        _fold(logits_sc[1 - p], i - 1, vt, max_sc, idx_sc)

    xs = x_ref[...]
    logits_sc[p] = jax.lax.dot_general(
        xs, emb_ref[...], (((1,), (1,)), ((), ())),
        preferred_element_type=jnp.float32)

    @pl.when(i == ng - 1)
    def _done():
        _fold(logits_sc[p], i, vt, max_sc, idx_sc)
        out_ref[...] = idx_sc[...]


def kernel(embedding, x, output_pos, temperature, topp, topk, embedding_bias=None):
    v, d = embedding.shape
    b, s, _ = x.shape
    vt = 4000
    assert v % vt == 0
    ng = v // vt

    xt = x.reshape(b, s * d)
    pos = output_pos.astype(jnp.int32)

    grid_spec = pltpu.PrefetchScalarGridSpec(
        num_scalar_prefetch=1,
        grid=(ng,),
        in_specs=[
            pl.BlockSpec((b, d), lambda i, pos_ref: (0, pos_ref[0])),
            pl.BlockSpec((vt, d), lambda i, pos_ref: (i, 0)),
        ],
        out_specs=pl.BlockSpec((b, 1), lambda i, pos_ref: (0, 0)),
        scratch_shapes=[
            pltpu.VMEM((2, b, vt), jnp.float32),
            pltpu.VMEM((b, 1), jnp.float32),
            pltpu.VMEM((b, 1), jnp.int32),
        ],
    )
    out = pl.pallas_call(
        functools.partial(_argmax_matmul_kernel, vt=vt, ng=ng),
        grid_spec=grid_spec,
        out_shape=jax.ShapeDtypeStruct((b, 1), jnp.int32),
        compiler_params=pltpu.CompilerParams(
            vmem_limit_bytes=100 * 1024 * 1024),
    )(pos, xt, embedding)
    return out[:, 0]


# speedup vs baseline: 1.0160x; 1.0034x over previous
import functools

import jax
import jax.numpy as jnp
from jax.experimental import pallas as pl
from jax.experimental.pallas import tpu as pltpu


def _fold(logits, tile_idx, vt, max_sc, idx_sc, enable=None):
    local_max = jnp.max(logits, axis=1, keepdims=True)
    local_idx = (jnp.argmax(logits, axis=1).astype(jnp.int32)[:, None]
                 + tile_idx * vt)
    better = local_max > max_sc[...]
    if enable is not None:
        better = jnp.logical_and(better, enable)
    idx_sc[...] = jnp.where(better, local_idx, idx_sc[...])
    max_sc[...] = jnp.where(better, local_max, max_sc[...])


def _argmax_matmul_kernel(pos_ref, x_ref, emb_ref, out_ref,
                          logits_sc, max_sc, idx_sc, *, vt: int, ng: int):
    i = pl.program_id(0)
    p = jax.lax.rem(i, 2)

    @pl.when(i == 0)
    def _init():
        max_sc[...] = jnp.full_like(max_sc[...], -jnp.inf)
        idx_sc[...] = jnp.zeros_like(idx_sc[...])

    _fold(logits_sc[1 - p], i - 1, vt, max_sc, idx_sc, enable=i > 0)

    xs = x_ref[...]
    logits_sc[p] = jax.lax.dot_general(
        xs, emb_ref[...], (((1,), (1,)), ((), ())),
        preferred_element_type=jnp.float32)

    @pl.when(i == ng - 1)
    def _done():
        _fold(logits_sc[p], i, vt, max_sc, idx_sc)
        out_ref[...] = idx_sc[...]


def kernel(embedding, x, output_pos, temperature, topp, topk, embedding_bias=None):
    v, d = embedding.shape
    b, s, _ = x.shape
    vt = 4000
    assert v % vt == 0
    ng = v // vt

    xt = x.reshape(b, s * d)
    pos = output_pos.astype(jnp.int32)

    grid_spec = pltpu.PrefetchScalarGridSpec(
        num_scalar_prefetch=1,
        grid=(ng,),
        in_specs=[
            pl.BlockSpec((b, d), lambda i, pos_ref: (0, pos_ref[0])),
            pl.BlockSpec((vt, d), lambda i, pos_ref: (i, 0)),
        ],
        out_specs=pl.BlockSpec((b, 1), lambda i, pos_ref: (0, 0)),
        scratch_shapes=[
            pltpu.VMEM((2, b, vt), jnp.float32),
            pltpu.VMEM((b, 1), jnp.float32),
            pltpu.VMEM((b, 1), jnp.int32),
        ],
    )
    out = pl.pallas_call(
        functools.partial(_argmax_matmul_kernel, vt=vt, ng=ng),
        grid_spec=grid_spec,
        out_shape=jax.ShapeDtypeStruct((b, 1), jnp.int32),
        compiler_params=pltpu.CompilerParams(
            vmem_limit_bytes=100 * 1024 * 1024),
    )(pos, xt, embedding)
    return out[:, 0]
